# Initial kernel scaffold; baseline (speedup 1.0000x reference)
import jax
import jax.numpy as jnp
from jax.experimental import pallas as pl


def kernel(x1, x2):
    raise NotImplementedError("write your pallas kernel here")


# speedup vs baseline: 40.5046x; 40.5046x over previous
import functools

import jax
import jax.numpy as jnp
import numpy as np
from jax import lax
from jax.experimental import pallas as pl
from jax.experimental.pallas import tpu as pltpu
from jax.experimental.pallas import tpu_sc as plsc

NBINS = 1000
NBINS_PAD = 1024
LANES = 16
NCORES = 2
NSUB = 16
NWORKERS = NCORES * NSUB
TOTAL = 8192 * 4096
PER_TILE = TOTAL // NWORKERS
CHUNK = 32768
NCHUNKS = PER_TILE // CHUNK
INV_WIDTH = float(np.float32(1.0) / np.float32((1.0 - 0.0) / NBINS))


def _sc_histograms(x1f, x2f):
    mesh = plsc.VectorSubcoreMesh(core_axis_name="c", subcore_axis_name="s")

    @functools.partial(
        pl.kernel,
        out_type=(
            jax.ShapeDtypeStruct((NWORKERS, NBINS_PAD), jnp.float32),
            jax.ShapeDtypeStruct((NWORKERS, NBINS_PAD), jnp.float32),
        ),
        mesh=mesh,
        compiler_params=pltpu.CompilerParams(needs_layout_passes=False),
        scratch_types=[
            pltpu.VMEM((CHUNK,), jnp.float32),
            pltpu.VMEM((LANES * NBINS_PAD,), jnp.float32),
            pltpu.VMEM((LANES * NBINS_PAD,), jnp.float32),
            pltpu.VMEM((NBINS_PAD,), jnp.float32),
        ],
    )
    def hist_kernel(x1_hbm, x2_hbm, o1_hbm, o2_hbm, buf, h1, h2, red):
        wid = lax.axis_index("s") * NCORES + lax.axis_index("c")
        base = wid * PER_TILE
        zeros16 = jnp.zeros((LANES,), jnp.float32)
        ones16 = jnp.ones((LANES,), jnp.float32)
        lane_off = lax.iota(jnp.int32, LANES) * NBINS_PAD
        scale = jnp.float32(INV_WIDTH)

        def zero_body(i, carry):
            h1[pl.ds(i * LANES, LANES)] = zeros16
            h2[pl.ds(i * LANES, LANES)] = zeros16
            return carry

        lax.fori_loop(0, LANES * NBINS_PAD // LANES, zero_body, 0)

        def process(x_hbm, h):
            def chunk_body(ci, carry):
                pltpu.sync_copy(x_hbm.at[pl.ds(base + ci * CHUNK, CHUNK)], buf)

                def vec_body(vi, c2):
                    v = buf[pl.ds(vi * LANES, LANES)]
                    idx = (v * scale).astype(jnp.int32)
                    idx = jnp.clip(idx, 0, NBINS - 1)
                    plsc.addupdate_scatter(h, [idx + lane_off], ones16)
                    return c2

                lax.fori_loop(0, CHUNK // LANES, vec_body, 0)
                return carry

            lax.fori_loop(0, NCHUNKS, chunk_body, 0)

        process(x1_hbm, h1)
        process(x2_hbm, h2)

        def reduce_write(h, o_hbm):
            def red_body(c, carry):
                acc = h[pl.ds(c * LANES, LANES)]
                for lane in range(1, LANES):
                    acc = acc + h[pl.ds(lane * NBINS_PAD + c * LANES, LANES)]
                red[pl.ds(c * LANES, LANES)] = acc
                return carry

            lax.fori_loop(0, NBINS_PAD // LANES, red_body, 0)
            pltpu.sync_copy(red, o_hbm.at[wid])

        reduce_write(h1, o1_hbm)
        reduce_write(h2, o2_hbm)

    return hist_kernel(x1f, x2f)


def _kl_body(c1_ref, c2_ref, out_ref):
    sz = jnp.float32(TOTAL)
    c1 = jnp.sum(c1_ref[...], axis=0, keepdims=True)
    c2 = jnp.sum(c2_ref[...], axis=0, keepdims=True)
    p = c1 / sz
    q = c2 / sz
    mask = (p > 0) & (q > 0)
    ps = jnp.where(mask, p, 1.0)
    qs = jnp.where(mask, q, 1.0)
    logp = jnp.log(ps)
    logq = jnp.log(qs)
    kl_fwd = jnp.sum(jnp.where(mask, p * (logp - logq), 0.0))
    kl_inv = jnp.sum(jnp.where(mask, q * (logq - logp), 0.0))
    out_ref[0, 0] = (kl_fwd + kl_inv) * 0.5


def _kl_from_partials(o1, o2):
    out = pl.pallas_call(
        _kl_body,
        out_shape=jax.ShapeDtypeStruct((1, 1), jnp.float32),
        out_specs=pl.BlockSpec(memory_space=pltpu.SMEM),
    )(o1, o2)
    return out[0, 0]


def kernel(x1, x2):
    o1, o2 = _sc_histograms(x1.reshape(-1), x2.reshape(-1))
    return _kl_from_partials(o1, o2)


# speedup vs baseline: 40.5631x; 1.0014x over previous
import functools

import jax
import jax.numpy as jnp
import numpy as np
from jax import lax
from jax.experimental import pallas as pl
from jax.experimental.pallas import tpu as pltpu
from jax.experimental.pallas import tpu_sc as plsc

NBINS = 1000
NBINS_PAD = 1024
LANE_STRIDE = NBINS_PAD + 1
LANES = 16
NCORES = 2
NSUB = 16
NWORKERS = NCORES * NSUB
TOTAL = 8192 * 4096
PER_TILE = TOTAL // NWORKERS
CHUNK = 32768
NCHUNKS = PER_TILE // CHUNK
INV_WIDTH = float(np.float32(1.0) / np.float32((1.0 - 0.0) / NBINS))


def _sc_histograms(x1f, x2f):
    mesh = plsc.VectorSubcoreMesh(core_axis_name="c", subcore_axis_name="s")

    @functools.partial(
        pl.kernel,
        out_type=(
            jax.ShapeDtypeStruct((NWORKERS, NBINS_PAD), jnp.float32),
            jax.ShapeDtypeStruct((NWORKERS, NBINS_PAD), jnp.float32),
        ),
        mesh=mesh,
        compiler_params=pltpu.CompilerParams(needs_layout_passes=False),
        scratch_types=[
            pltpu.VMEM((CHUNK,), jnp.float32),
            pltpu.VMEM((LANES * LANE_STRIDE,), jnp.float32),
            pltpu.VMEM((LANES * LANE_STRIDE,), jnp.float32),
            pltpu.VMEM((NBINS_PAD,), jnp.float32),
        ],
    )
    def hist_kernel(x1_hbm, x2_hbm, o1_hbm, o2_hbm, buf, h1, h2, red):
        wid = lax.axis_index("s") * NCORES + lax.axis_index("c")
        base = wid * PER_TILE
        zeros16 = jnp.zeros((LANES,), jnp.float32)
        ones16 = jnp.ones((LANES,), jnp.float32)
        lane_off = lax.iota(jnp.int32, LANES) * LANE_STRIDE
        scale = jnp.float32(INV_WIDTH)

        def zero_body(i, carry):
            h1[pl.ds(i * LANES, LANES)] = zeros16
            h2[pl.ds(i * LANES, LANES)] = zeros16
            return carry

        lax.fori_loop(0, LANES * LANE_STRIDE // LANES, zero_body, 0)

        def process(x_hbm, h):
            def chunk_body(ci, carry):
                pltpu.sync_copy(x_hbm.at[pl.ds(base + ci * CHUNK, CHUNK)], buf)

                def vec_body(vi, c2):
                    v = buf[pl.ds(vi * LANES, LANES)]
                    idx = (v * scale).astype(jnp.int32)
                    idx = jnp.clip(idx, 0, NBINS - 1)
                    plsc.addupdate_scatter(h, [idx + lane_off], ones16)
                    return c2

                lax.fori_loop(0, CHUNK // LANES, vec_body, 0)
                return carry

            lax.fori_loop(0, NCHUNKS, chunk_body, 0)

        process(x1_hbm, h1)
        process(x2_hbm, h2)

        def reduce_write(h, o_hbm):
            def red_body(c, carry):
                acc = h[pl.ds(c * LANES, LANES)]
                for lane in range(1, LANES):
                    acc = acc + h[pl.ds(lane * LANE_STRIDE + c * LANES, LANES)]
                red[pl.ds(c * LANES, LANES)] = acc
                return carry

            lax.fori_loop(0, NBINS_PAD // LANES, red_body, 0)
            pltpu.sync_copy(red, o_hbm.at[wid])

        reduce_write(h1, o1_hbm)
        reduce_write(h2, o2_hbm)

    return hist_kernel(x1f, x2f)


def _kl_body(c1_ref, c2_ref, out_ref):
    sz = jnp.float32(TOTAL)
    c1 = jnp.sum(c1_ref[...], axis=0, keepdims=True)
    c2 = jnp.sum(c2_ref[...], axis=0, keepdims=True)
    p = c1 / sz
    q = c2 / sz
    mask = (p > 0) & (q > 0)
    ps = jnp.where(mask, p, 1.0)
    qs = jnp.where(mask, q, 1.0)
    logp = jnp.log(ps)
    logq = jnp.log(qs)
    kl_fwd = jnp.sum(jnp.where(mask, p * (logp - logq), 0.0))
    kl_inv = jnp.sum(jnp.where(mask, q * (logq - logp), 0.0))
    out_ref[0, 0] = (kl_fwd + kl_inv) * 0.5


def _kl_from_partials(o1, o2):
    out = pl.pallas_call(
        _kl_body,
        out_shape=jax.ShapeDtypeStruct((1, 1), jnp.float32),
        out_specs=pl.BlockSpec(memory_space=pltpu.SMEM),
    )(o1, o2)
    return out[0, 0]


def kernel(x1, x2):
    o1, o2 = _sc_histograms(x1.reshape(-1), x2.reshape(-1))
    return _kl_from_partials(o1, o2)


# speedup vs baseline: 264.4709x; 6.5200x over previous
import functools

import jax
import jax.numpy as jnp
import numpy as np
from jax import lax
from jax.experimental import pallas as pl
from jax.experimental.pallas import tpu as pltpu
from jax.experimental.pallas import tpu_sc as plsc

NBINS = 1000
NBINS_PAD = 1024
LANE_STRIDE = NBINS_PAD + 1
LANES = 16
NCORES = 2
NSUB = 16
NWORKERS = NCORES * NSUB
ROWS = 8192
COLS = 4096
TOTAL = ROWS * COLS
ROWS_PER_TILE = ROWS // NWORKERS
SLAB = 8
NSLABS = ROWS_PER_TILE // SLAB
VECS_PER_SLAB = SLAB * COLS // LANES
UNROLL = 8
INV_WIDTH = float(np.float32(1.0) / np.float32((1.0 - 0.0) / NBINS))


def _sc_histograms(x1, x2):
    mesh = plsc.VectorSubcoreMesh(core_axis_name="c", subcore_axis_name="s")

    @functools.partial(
        pl.kernel,
        out_type=(
            jax.ShapeDtypeStruct((NWORKERS * NBINS_PAD,), jnp.float32),
            jax.ShapeDtypeStruct((NWORKERS * NBINS_PAD,), jnp.float32),
        ),
        mesh=mesh,
        compiler_params=pltpu.CompilerParams(
            needs_layout_passes=False, use_tc_tiling_on_sc=True),
        scratch_types=[
            pltpu.VMEM((SLAB, COLS), jnp.float32),
            pltpu.VMEM((SLAB, COLS), jnp.float32),
            pltpu.VMEM((LANES * LANE_STRIDE,), jnp.float32),
            pltpu.VMEM((NBINS_PAD,), jnp.float32),
            pltpu.SemaphoreType.DMA,
            pltpu.SemaphoreType.DMA,
        ],
    )
    def hist_kernel(x1_hbm, x2_hbm, o1_hbm, o2_hbm, buf_a, buf_b, h, red,
                    sem_a, sem_b):
        bufs = (buf_a, buf_b)
        sems = (sem_a, sem_b)
        wid = lax.axis_index("s") * NCORES + lax.axis_index("c")
        row_base = wid * ROWS_PER_TILE
        zeros16 = jnp.zeros((LANES,), jnp.float32)
        ones16 = jnp.ones((LANES,), jnp.float32)
        lane_off = lax.iota(jnp.int32, LANES) * LANE_STRIDE
        scale = jnp.float32(INV_WIDTH)

        def zero_hist():
            def zero_body(i, carry):
                h[pl.ds(i * LANES, LANES)] = zeros16
                return carry

            lax.fori_loop(0, LANES * LANE_STRIDE // LANES, zero_body, 0)

        def process(x_hbm):
            def copy_handle(si, b):
                return pltpu.make_async_copy(
                    x_hbm.at[pl.ds(row_base + si * SLAB, SLAB), :],
                    bufs[b], sems[b])

            def consume(buf):
                @plsc.parallel_loop(0, VECS_PER_SLAB, step=1, unroll=UNROLL)
                def vec_body(vi):
                    r = lax.shift_right_logical(vi, 8)
                    c = lax.shift_left(lax.bitwise_and(vi, 255), 4)
                    v = buf[r, pl.ds(c, LANES)]
                    idx = (v * scale).astype(jnp.int32)
                    idx_u = lax.bitcast_convert_type(idx, jnp.uint32)
                    idx_u = lax.min(idx_u, jnp.uint32(NBINS - 1))
                    idx = lax.bitcast_convert_type(idx_u, jnp.int32)
                    plsc.addupdate_scatter(h, [idx + lane_off], ones16)

            copy_handle(0, 0).start()
            copy_handle(1, 1).start()

            def slab_body(ci, carry):
                for b in range(2):
                    cur = ci * 2 + b
                    copy_handle(cur, b).wait()
                    consume(bufs[b])
                    nxt = cur + 2

                    @pl.when(nxt < NSLABS)
                    def _start_next():
                        copy_handle(nxt, b).start()
                return carry

            lax.fori_loop(0, NSLABS // 2, slab_body, 0)

        def reduce_write(o_hbm):
            def red_body(c, carry):
                acc = h[pl.ds(c * LANES, LANES)]
                for lane in range(1, LANES):
                    acc = acc + h[pl.ds(lane * LANE_STRIDE + c * LANES, LANES)]
                red[pl.ds(c * LANES, LANES)] = acc
                return carry

            lax.fori_loop(0, NBINS_PAD // LANES, red_body, 0)
            pltpu.sync_copy(red, o_hbm.at[pl.ds(wid * NBINS_PAD, NBINS_PAD)])

        zero_hist()
        process(x1_hbm)
        reduce_write(o1_hbm)
        zero_hist()
        process(x2_hbm)
        reduce_write(o2_hbm)

    return hist_kernel(x1, x2)


def _kl_body(c1_ref, c2_ref, out_ref):
    sz = jnp.float32(TOTAL)
    c1 = jnp.sum(c1_ref[...], axis=0, keepdims=True)
    c2 = jnp.sum(c2_ref[...], axis=0, keepdims=True)
    p = c1 / sz
    q = c2 / sz
    mask = (p > 0) & (q > 0)
    ps = jnp.where(mask, p, 1.0)
    qs = jnp.where(mask, q, 1.0)
    logp = jnp.log(ps)
    logq = jnp.log(qs)
    kl_fwd = jnp.sum(jnp.where(mask, p * (logp - logq), 0.0))
    kl_inv = jnp.sum(jnp.where(mask, q * (logq - logp), 0.0))
    out_ref[0, 0] = (kl_fwd + kl_inv) * 0.5


def _kl_from_partials(o1, o2):
    out = pl.pallas_call(
        _kl_body,
        out_shape=jax.ShapeDtypeStruct((1, 1), jnp.float32),
        out_specs=pl.BlockSpec(memory_space=pltpu.SMEM),
    )(o1, o2)
    return out[0, 0]


def kernel(x1, x2):
    o1, o2 = _sc_histograms(x1, x2)
    return _kl_from_partials(o1.reshape(NWORKERS, NBINS_PAD),
                             o2.reshape(NWORKERS, NBINS_PAD))


# speedup vs baseline: 276.9948x; 1.0474x over previous
import functools

import jax
import jax.numpy as jnp
import numpy as np
from jax import lax
from jax.experimental import pallas as pl
from jax.experimental.pallas import tpu as pltpu
from jax.experimental.pallas import tpu_sc as plsc

NBINS = 1000
NBINS_PAD = 1024
LANES = 16
NCORES = 2
NSUB = 16
NWORKERS = NCORES * NSUB
ROWS = 8192
COLS = 4096
TOTAL = ROWS * COLS
ROWS_PER_TILE = ROWS // NWORKERS
SLAB = 8
NSLABS = ROWS_PER_TILE // SLAB
VECS_PER_SLAB = SLAB * COLS // LANES
UNROLL = 16
INV_WIDTH = float(np.float32(1.0) / np.float32((1.0 - 0.0) / NBINS))


def _sc_histograms(x1, x2):
    mesh = plsc.VectorSubcoreMesh(core_axis_name="c", subcore_axis_name="s")

    @functools.partial(
        pl.kernel,
        out_type=(
            jax.ShapeDtypeStruct((NWORKERS * NBINS_PAD,), jnp.float32),
            jax.ShapeDtypeStruct((NWORKERS * NBINS_PAD,), jnp.float32),
        ),
        mesh=mesh,
        compiler_params=pltpu.CompilerParams(
            needs_layout_passes=False, use_tc_tiling_on_sc=True),
        scratch_types=[
            pltpu.VMEM((SLAB, COLS), jnp.float32),
            pltpu.VMEM((SLAB, COLS), jnp.float32),
            pltpu.VMEM((NBINS_PAD,), jnp.float32),
            pltpu.SemaphoreType.DMA,
            pltpu.SemaphoreType.DMA,
        ],
    )
    def hist_kernel(x1_hbm, x2_hbm, o1_hbm, o2_hbm, buf_a, buf_b, h,
                    sem_a, sem_b):
        bufs = (buf_a, buf_b)
        sems = (sem_a, sem_b)
        wid = lax.axis_index("s") * NCORES + lax.axis_index("c")
        row_base = wid * ROWS_PER_TILE
        zeros16 = jnp.zeros((LANES,), jnp.float32)
        ones16 = jnp.ones((LANES,), jnp.float32)
        scale = jnp.float32(INV_WIDTH)

        def zero_hist():
            def zero_body(i, carry):
                h[pl.ds(i * LANES, LANES)] = zeros16
                return carry

            lax.fori_loop(0, NBINS_PAD // LANES, zero_body, 0)

        def process(x_hbm):
            def copy_handle(si, b):
                return pltpu.make_async_copy(
                    x_hbm.at[pl.ds(row_base + si * SLAB, SLAB), :],
                    bufs[b], sems[b])

            def consume(buf):
                @plsc.parallel_loop(0, VECS_PER_SLAB, step=1, unroll=UNROLL)
                def vec_body(vi):
                    r = lax.shift_right_logical(vi, 8)
                    c = lax.shift_left(lax.bitwise_and(vi, 255), 4)
                    v = buf[r, pl.ds(c, LANES)]
                    idx = (v * scale).astype(jnp.int32)
                    idx_u = lax.bitcast_convert_type(idx, jnp.uint32)
                    idx_u = lax.min(idx_u, jnp.uint32(NBINS - 1))
                    idx = lax.bitcast_convert_type(idx_u, jnp.int32)
                    plsc.addupdate_scatter(h, [idx], ones16)

            copy_handle(0, 0).start()
            copy_handle(1, 1).start()

            def slab_body(ci, carry):
                for b in range(2):
                    cur = ci * 2 + b
                    copy_handle(cur, b).wait()
                    consume(bufs[b])
                    nxt = cur + 2

                    @pl.when(nxt < NSLABS)
                    def _start_next():
                        copy_handle(nxt, b).start()
                return carry

            lax.fori_loop(0, NSLABS // 2, slab_body, 0)

        def reduce_write(o_hbm):
            pltpu.sync_copy(h, o_hbm.at[pl.ds(wid * NBINS_PAD, NBINS_PAD)])

        zero_hist()
        process(x1_hbm)
        reduce_write(o1_hbm)
        zero_hist()
        process(x2_hbm)
        reduce_write(o2_hbm)

    return hist_kernel(x1, x2)


def _kl_body(c1_ref, c2_ref, out_ref):
    sz = jnp.float32(TOTAL)
    c1 = jnp.sum(c1_ref[...], axis=0, keepdims=True)
    c2 = jnp.sum(c2_ref[...], axis=0, keepdims=True)
    p = c1 / sz
    q = c2 / sz
    mask = (p > 0) & (q > 0)
    ps = jnp.where(mask, p, 1.0)
    qs = jnp.where(mask, q, 1.0)
    logp = jnp.log(ps)
    logq = jnp.log(qs)
    kl_fwd = jnp.sum(jnp.where(mask, p * (logp - logq), 0.0))
    kl_inv = jnp.sum(jnp.where(mask, q * (logq - logp), 0.0))
    out_ref[0, 0] = (kl_fwd + kl_inv) * 0.5


def _kl_from_partials(o1, o2):
    out = pl.pallas_call(
        _kl_body,
        out_shape=jax.ShapeDtypeStruct((1, 1), jnp.float32),
        out_specs=pl.BlockSpec(memory_space=pltpu.SMEM),
    )(o1, o2)
    return out[0, 0]


def kernel(x1, x2):
    o1, o2 = _sc_histograms(x1, x2)
    return _kl_from_partials(o1.reshape(NWORKERS, NBINS_PAD),
                             o2.reshape(NWORKERS, NBINS_PAD))
